# baseline (speedup 1.0000x reference)
import functools

import numpy as np

import jax
import jax.numpy as jnp
from jax import lax
from jax.experimental import pallas as pl
from jax.experimental.pallas import tpu as pltpu
from jax.experimental.pallas import tpu_sc as plsc

N = 10000
E = 320000
D = 128
DE = 16
H = 128
HW = H // 2

_info = plsc.get_sparse_core_info()
_NC = _info.num_cores
_NS = _info.num_subcores
_NW = _NC * _NS
_EPW = E // _NW
_C = 80
_NCH = _EPW // _C


def _proj_body(src_ref, dst_ref, ws_ref, wd_ref, ps_ref, pd_ref):
    ps_ref[...] = jnp.dot(src_ref[...], ws_ref[...],
                          preferred_element_type=jnp.float32)
    pd_ref[...] = jnp.dot(dst_ref[...], wd_ref[...],
                          preferred_element_type=jnp.float32)


def _project(src, dst, ws, wd):
    bn = 2000
    grid = N // bn
    return pl.pallas_call(
        _proj_body,
        grid=(grid,),
        in_specs=[
            pl.BlockSpec((bn, D), lambda i: (i, 0)),
            pl.BlockSpec((bn, D), lambda i: (i, 0)),
            pl.BlockSpec((D, H), lambda i: (0, 0)),
            pl.BlockSpec((D, H), lambda i: (0, 0)),
        ],
        out_specs=[
            pl.BlockSpec((bn, H), lambda i: (i, 0)),
            pl.BlockSpec((bn, H), lambda i: (i, 0)),
        ],
        out_shape=[
            jax.ShapeDtypeStruct((N, H), jnp.float32),
            jax.ShapeDtypeStruct((N, H), jnp.float32),
        ],
    )(src, dst, ws, wd)


@functools.partial(
    pl.kernel,
    out_type=jax.ShapeDtypeStruct((E // 2, H), jnp.float32),
    mesh=plsc.VectorSubcoreMesh(core_axis_name="c", subcore_axis_name="s"),
    scratch_types=[
        pltpu.VMEM((_C,), jnp.int32),
        pltpu.VMEM((_C,), jnp.int32),
        pltpu.VMEM((_C,), jnp.int32),
        pltpu.VMEM((_C,), jnp.int32),
        pltpu.VMEM((_C, H), jnp.float32),
        pltpu.VMEM((_C, H), jnp.float32),
        pltpu.VMEM((_C, H), jnp.float32),
        pltpu.VMEM((_C, H), jnp.float32),
        pltpu.SemaphoreType.DMA,
        pltpu.SemaphoreType.DMA,
        pltpu.SemaphoreType.DMA,
        pltpu.SemaphoreType.DMA,
        pltpu.SemaphoreType.DMA,
        pltpu.SemaphoreType.DMA,
    ],
)
def _sc_gather_sum(ps_hbm, pd_hbm, si_hbm, di_hbm, out_hbm,
                   idx_s0, idx_d0, idx_s1, idx_d1,
                   buf_a0, buf_b0, buf_a1, buf_b1,
                   sem_g0, sem_g1, sem_o0, sem_o1,
                   sem_i0, sem_i1):
    wid = lax.axis_index("s") * _NC + lax.axis_index("c")
    base = wid * _EPW
    base2 = wid * (_EPW // 2)
    idxs_s = (idx_s0, idx_s1)
    idxs_d = (idx_d0, idx_d1)
    bufs_a = (buf_a0, buf_a1)
    bufs_b = (buf_b0, buf_b1)
    sems_g = (sem_g0, sem_g1)
    sems_o = (sem_o0, sem_o1)
    sems_i = (sem_i0, sem_i1)

    def fire_idx(ci, b):
        off = base + ci * _C
        pltpu.async_copy(si_hbm.at[pl.ds(off, _C)], idxs_s[b], sems_i[b])
        pltpu.async_copy(di_hbm.at[pl.ds(off, _C)], idxs_d[b], sems_i[b])

    def drain_idx(b):
        pltpu.make_async_copy(si_hbm.at[pl.ds(0, _C)], idxs_s[b],
                              sems_i[b]).wait()
        pltpu.make_async_copy(si_hbm.at[pl.ds(0, _C)], idxs_d[b],
                              sems_i[b]).wait()

    def fire(ci, b):
        pltpu.async_copy(ps_hbm.at[idxs_s[b]], bufs_a[b], sems_g[b])
        pltpu.async_copy(pd_hbm.at[idxs_d[b]], bufs_b[b], sems_g[b])

    def drain_gather(b):
        pltpu.make_async_copy(ps_hbm.at[pl.ds(0, _C)], bufs_a[b],
                              sems_g[b]).wait()
        pltpu.make_async_copy(ps_hbm.at[pl.ds(0, _C)], bufs_b[b],
                              sems_g[b]).wait()

    def add_rows(b):
        ba, bb = bufs_a[b], bufs_b[b]
        mhi = jnp.int32(-65536)
        rnd = jnp.int32(0x8000)

        def row_body(r, c2):
            for rr in range(2):
                row = 2 * r + rr
                half = rr * (H // 2)
                for j in range(HW // 16):
                    slo = pl.ds(j * 32, 16)
                    shi = pl.ds(j * 32 + 16, 16)
                    se = ba[row, slo] + bb[row, slo]
                    so = ba[row, shi] + bb[row, shi]
                    ie = lax.bitcast_convert_type(se, jnp.int32)
                    io = lax.bitcast_convert_type(so, jnp.int32)
                    lo = lax.shift_right_logical(ie + rnd, 16)
                    hi = (io + rnd) & mhi
                    ba[r, pl.ds(half + j * 16, 16)] = \
                        lax.bitcast_convert_type(hi | lo, jnp.float32)
            return c2

        lax.fori_loop(0, _C // 2, row_body, 0)

    def start_out(ci, b):
        off2 = pl.multiple_of(base2 + ci * (_C // 2), 8)
        pltpu.async_copy(
            bufs_a[b].at[pl.ds(0, _C // 2)],
            out_hbm.at[pl.ds(off2, _C // 2)],
            sems_o[b])

    def drain_out(b):
        pltpu.make_async_copy(ps_hbm.at[pl.ds(0, _C // 2)],
                              bufs_a[b].at[pl.ds(0, _C // 2)],
                              sems_o[b]).wait()

    fire_idx(0, 0)
    fire_idx(1, 1)
    drain_idx(0)
    fire(0, 0)
    drain_idx(1)
    fire(1, 1)

    def pair_body(g, carry):
        ci0 = 2 * g
        for b in range(2):
            ci = ci0 + b
            drain_gather(b)

            @pl.when(ci + 2 < _NCH)
            def _():
                fire_idx(ci + 2, b)

            add_rows(b)
            start_out(ci, b)

            @pl.when(ci + 2 < _NCH)
            def _():
                drain_idx(b)
                drain_out(b)
                fire(ci + 2, b)
        return carry

    lax.fori_loop(0, _NCH // 2, pair_body, 0)
    drain_gather(0)
    add_rows(0)
    start_out(_NCH - 1, 0)
    drain_out(0)
    drain_out(1)


def _tail_body(pre_ref, ef_ref, we_ref, b1_ref, w2_ref, b2_ref,
               g_ref, bt_ref, out_ref):
    ef = ef_ref[...]
    x = pre_ref[...].astype(jnp.float32) + b1_ref[...]
    x = x + jnp.dot(ef, we_ref[...], preferred_element_type=jnp.float32)
    h = x * (1.0 / (1.0 + jnp.exp(-x)))
    y = jnp.dot(h, w2_ref[...], preferred_element_type=jnp.float32) + b2_ref[...]
    mu = jnp.mean(y, axis=-1, keepdims=True)
    var = jnp.mean((y - mu) * (y - mu), axis=-1, keepdims=True)
    yn = (y - mu) * lax.rsqrt(var + 1e-5) * g_ref[...] + bt_ref[...]
    out_ref[...] = yn + ef


def _tail(pre, ef, we, b1, w2, b2, gamma, beta):
    be = 2000
    grid = E // be
    return pl.pallas_call(
        _tail_body,
        grid=(grid,),
        in_specs=[
            pl.BlockSpec((be, H), lambda i: (i, 0)),
            pl.BlockSpec((be, DE), lambda i: (i, 0)),
            pl.BlockSpec((DE, H), lambda i: (0, 0)),
            pl.BlockSpec((1, H), lambda i: (0, 0)),
            pl.BlockSpec((H, DE), lambda i: (0, 0)),
            pl.BlockSpec((1, DE), lambda i: (0, 0)),
            pl.BlockSpec((1, DE), lambda i: (0, 0)),
            pl.BlockSpec((1, DE), lambda i: (0, 0)),
        ],
        out_specs=pl.BlockSpec((be, DE), lambda i: (i, 0)),
        out_shape=jax.ShapeDtypeStruct((E, DE), jnp.float32),
    )(pre, ef, we, b1, w2, b2, gamma, beta)


def _unpack_bf16(x):
    n, hw = x.shape
    return lax.bitcast_convert_type(x, jnp.bfloat16).reshape(2 * n, hw)


_PERM = np.empty((H,), dtype=np.int32)
for _w in range(H // 2):
    _j, _t = divmod(_w, 16)
    _PERM[2 * _w] = 32 * _j + _t
    _PERM[2 * _w + 1] = 32 * _j + 16 + _t


def kernel(src_node_features, dst_node_features, edge_features,
           src_indices, dst_indices, W1, b1, W2, b2, gamma, beta):
    ws = W1[:D]
    wd = W1[D:2 * D]
    we = W1[2 * D:]
    si = src_indices.astype(jnp.int32)
    di = dst_indices.astype(jnp.int32)
    ps, pd = _project(src_node_features, dst_node_features, ws, wd)
    pre_packed = _sc_gather_sum(ps, pd, si, di)
    pre = _unpack_bf16(pre_packed)
    return _tail(pre, edge_features, we[:, _PERM],
                 b1[_PERM].reshape(1, H), W2[_PERM], b2.reshape(1, DE),
                 gamma.reshape(1, DE), beta.reshape(1, DE))


# speedup vs baseline: 32.1366x; 32.1366x over previous
import functools

import numpy as np

import jax
import jax.numpy as jnp
from jax import lax
from jax.experimental import pallas as pl
from jax.experimental.pallas import tpu as pltpu
from jax.experimental.pallas import tpu_sc as plsc

N = 10000
E = 320000
D = 128
DE = 16
H = 128

_info = plsc.get_sparse_core_info()
_NC = _info.num_cores
_NS = _info.num_subcores
_NW = _NC * _NS
_EPW = E // _NW
_C = 80
_CH = _C // 2
_NCH = _EPW // _C

_MHI = np.int32(-65536)
_RND = np.int32(0x8000)


def _proj_body(src_ref, dst_ref, ws_ref, wd_ref, ps_ref, pd_ref):
    ps_ref[...] = jnp.dot(src_ref[...], ws_ref[...],
                          preferred_element_type=jnp.float32)
    pd_ref[...] = jnp.dot(dst_ref[...], wd_ref[...],
                          preferred_element_type=jnp.float32)


def _project(src, dst, ws, wd):
    bn = 2000
    return pl.pallas_call(
        _proj_body,
        grid=(N // bn,),
        in_specs=[
            pl.BlockSpec((bn, D), lambda i: (i, 0)),
            pl.BlockSpec((bn, D), lambda i: (i, 0)),
            pl.BlockSpec((D, H), lambda i: (0, 0)),
            pl.BlockSpec((D, H), lambda i: (0, 0)),
        ],
        out_specs=[
            pl.BlockSpec((bn, H), lambda i: (i, 0)),
            pl.BlockSpec((bn, H), lambda i: (i, 0)),
        ],
        out_shape=[
            jax.ShapeDtypeStruct((N, H), jnp.float32),
            jax.ShapeDtypeStruct((N, H), jnp.float32),
        ],
    )(src, dst, ws, wd)


@functools.partial(
    pl.kernel,
    out_type=jax.ShapeDtypeStruct((E // 2, H), jnp.float32),
    mesh=plsc.VectorSubcoreMesh(core_axis_name="c", subcore_axis_name="s"),
    scratch_types=[
        pltpu.VMEM((_C,), jnp.int32),
        pltpu.VMEM((_C,), jnp.int32),
        pltpu.VMEM((_C,), jnp.int32),
        pltpu.VMEM((_C,), jnp.int32),
        pltpu.VMEM((_C, H), jnp.float32),
        pltpu.VMEM((_C, H), jnp.float32),
        pltpu.VMEM((_C, H), jnp.float32),
        pltpu.VMEM((_C, H), jnp.float32),
        pltpu.SemaphoreType.DMA,
        pltpu.SemaphoreType.DMA,
        pltpu.SemaphoreType.DMA,
        pltpu.SemaphoreType.DMA,
        pltpu.SemaphoreType.DMA,
        pltpu.SemaphoreType.DMA,
    ],
)
def _sc_gather_sum(ps_hbm, pd_hbm, si_hbm, di_hbm, out_hbm,
                   idx_s0, idx_d0, idx_s1, idx_d1,
                   buf_a0, buf_b0, buf_a1, buf_b1,
                   sem_g0, sem_g1, sem_o0, sem_o1,
                   sem_i0, sem_i1):
    wid = lax.axis_index("s") * _NC + lax.axis_index("c")
    base = wid * _EPW
    base2 = wid * (_EPW // 2)
    idxs_s = (idx_s0, idx_s1)
    idxs_d = (idx_d0, idx_d1)
    bufs_a = (buf_a0, buf_a1)
    bufs_b = (buf_b0, buf_b1)
    sems_g = (sem_g0, sem_g1)
    sems_o = (sem_o0, sem_o1)
    sems_i = (sem_i0, sem_i1)

    def fire_idx(ci, b):
        off = base + ci * _C
        pltpu.async_copy(si_hbm.at[pl.ds(off, _C)], idxs_s[b], sems_i[b])
        pltpu.async_copy(di_hbm.at[pl.ds(off, _C)], idxs_d[b], sems_i[b])

    def drain_idx(b):
        pltpu.make_async_copy(si_hbm.at[pl.ds(0, _C)], idxs_s[b],
                              sems_i[b]).wait()
        pltpu.make_async_copy(si_hbm.at[pl.ds(0, _C)], idxs_d[b],
                              sems_i[b]).wait()

    def fire(ci, b):
        pltpu.async_copy(ps_hbm.at[idxs_s[b]], bufs_a[b], sems_g[b])
        pltpu.async_copy(pd_hbm.at[idxs_d[b]], bufs_b[b], sems_g[b])

    def drain_gather(b):
        pltpu.make_async_copy(ps_hbm.at[pl.ds(0, _C)], bufs_a[b],
                              sems_g[b]).wait()
        pltpu.make_async_copy(ps_hbm.at[pl.ds(0, _C)], bufs_b[b],
                              sems_g[b]).wait()

    def add_rows(b):
        ba, bb = bufs_a[b], bufs_b[b]

        def pack_row(src_row, dst_row, dst_col):
            for j in range(H // 32):
                slo = pl.ds(j * 32, 16)
                shi = pl.ds(j * 32 + 16, 16)
                se = ba[src_row, slo] + bb[src_row, slo]
                so = ba[src_row, shi] + bb[src_row, shi]
                ie = lax.bitcast_convert_type(se, jnp.int32)
                io = lax.bitcast_convert_type(so, jnp.int32)
                lo = lax.shift_right_logical(ie + _RND, 16)
                hi = (io + _RND) & _MHI
                ba[dst_row, pl.ds(dst_col + j * 16, 16)] = \
                    lax.bitcast_convert_type(hi | lo, jnp.float32)

        def row_body(r, c2):
            pack_row(r, r, 0)
            return c2

        def row_body2(r, c2):
            pack_row(r + _CH, r, H // 2)
            return c2

        lax.fori_loop(0, _CH, row_body, 0)
        lax.fori_loop(0, _CH, row_body2, 0)

    def start_out(ci, b):
        off2 = pl.multiple_of(base2 + ci * _CH, 8)
        pltpu.async_copy(bufs_a[b].at[pl.ds(0, _CH)],
                         out_hbm.at[pl.ds(off2, _CH)],
                         sems_o[b])

    def drain_out(b):
        pltpu.make_async_copy(ps_hbm.at[pl.ds(0, _CH)],
                              bufs_a[b].at[pl.ds(0, _CH)],
                              sems_o[b]).wait()

    fire_idx(0, 0)
    fire_idx(1, 1)
    drain_idx(0)
    fire(0, 0)
    drain_idx(1)
    fire(1, 1)

    def pair_body(g, carry):
        ci0 = 2 * g
        for b in range(2):
            ci = ci0 + b
            drain_gather(b)

            @pl.when(ci + 2 < _NCH)
            def _():
                fire_idx(ci + 2, b)

            add_rows(b)
            start_out(ci, b)

            @pl.when(ci + 2 < _NCH)
            def _():
                drain_idx(b)
                drain_out(b)
                fire(ci + 2, b)
        return carry

    lax.fori_loop(0, _NCH // 2, pair_body, 0)
    drain_gather(0)
    add_rows(0)
    start_out(_NCH - 1, 0)
    drain_out(0)
    drain_out(1)


_BP = 1000
_GP = _BP // _CH


def _mlp_ln(x, ef, we, b1, w2, b2, g, bt):
    x = x + b1 + jnp.dot(ef, we, preferred_element_type=jnp.float32)
    h = x * (1.0 / (1.0 + jnp.exp(-x)))
    y = jnp.dot(h, w2, preferred_element_type=jnp.float32) + b2
    mu = jnp.mean(y, axis=-1, keepdims=True)
    var = jnp.mean((y - mu) * (y - mu), axis=-1, keepdims=True)
    return (y - mu) * lax.rsqrt(var + 1e-5) * g + bt + ef


def _tail_body(pk_ref, ef_ref, we_ref, b1_ref, w2_ref, b2_ref,
               g_ref, bt_ref, out_ref):
    wi = lax.bitcast_convert_type(pk_ref[...], jnp.int32)
    lo = lax.bitcast_convert_type(wi << 16, jnp.float32)
    hi = lax.bitcast_convert_type(wi & _MHI, jnp.float32)
    xf = jnp.concatenate([lo[:, :H // 2], hi[:, :H // 2]], axis=1)
    xs = jnp.concatenate([lo[:, H // 2:], hi[:, H // 2:]], axis=1)
    ef_blk = ef_ref[...]
    ef_f = jnp.concatenate(
        [ef_blk[2 * _CH * g:2 * _CH * g + _CH] for g in range(_GP)], axis=0)
    ef_s = jnp.concatenate(
        [ef_blk[2 * _CH * g + _CH:2 * _CH * (g + 1)] for g in range(_GP)],
        axis=0)
    args = (we_ref[...], b1_ref[...], w2_ref[...], b2_ref[...],
            g_ref[...], bt_ref[...])
    yf = _mlp_ln(xf, ef_f, *args)
    ys = _mlp_ln(xs, ef_s, *args)
    for g in range(_GP):
        out_ref[pl.ds(2 * _CH * g, _CH), :] = yf[_CH * g:_CH * (g + 1)]
        out_ref[pl.ds(2 * _CH * g + _CH, _CH), :] = ys[_CH * g:_CH * (g + 1)]


def _tail(pk, ef, we, b1, w2, b2, gamma, beta):
    nb = (E // 2) // _BP
    return pl.pallas_call(
        _tail_body,
        grid=(nb,),
        in_specs=[
            pl.BlockSpec((_BP, H), lambda i: (i, 0)),
            pl.BlockSpec((2 * _BP, DE), lambda i: (i, 0)),
            pl.BlockSpec((DE, H), lambda i: (0, 0)),
            pl.BlockSpec((1, H), lambda i: (0, 0)),
            pl.BlockSpec((H, DE), lambda i: (0, 0)),
            pl.BlockSpec((1, DE), lambda i: (0, 0)),
            pl.BlockSpec((1, DE), lambda i: (0, 0)),
            pl.BlockSpec((1, DE), lambda i: (0, 0)),
        ],
        out_specs=pl.BlockSpec((2 * _BP, DE), lambda i: (i, 0)),
        out_shape=jax.ShapeDtypeStruct((E, DE), jnp.float32),
    )(pk, ef, we, b1, w2, b2, gamma, beta)


_PERM = np.empty((H,), dtype=np.int32)
for _c in range(H // 2):
    _PERM[_c] = 32 * (_c // 16) + (_c % 16)
    _PERM[_c + H // 2] = _PERM[_c] + 16


def kernel(src_node_features, dst_node_features, edge_features,
           src_indices, dst_indices, W1, b1, W2, b2, gamma, beta):
    ws = W1[:D]
    wd = W1[D:2 * D]
    we = W1[2 * D:]
    si = src_indices.astype(jnp.int32)
    di = dst_indices.astype(jnp.int32)
    ps, pd = _project(src_node_features, dst_node_features, ws, wd)
    pk = _sc_gather_sum(ps, pd, si, di)
    return _tail(pk, edge_features, we[:, _PERM],
                 b1[_PERM].reshape(1, H), W2[_PERM], b2.reshape(1, DE),
                 gamma.reshape(1, DE), beta.reshape(1, DE))


# speedup vs baseline: 37.0804x; 1.1538x over previous
import functools

import numpy as np

import jax
import jax.numpy as jnp
from jax import lax
from jax.experimental import pallas as pl
from jax.experimental.pallas import tpu as pltpu
from jax.experimental.pallas import tpu_sc as plsc

N = 10000
E = 320000
D = 128
DE = 16
H = 128

_info = plsc.get_sparse_core_info()
_NC = _info.num_cores
_NS = _info.num_subcores
_NW = _NC * _NS
_EPW = E // _NW
_C = 80
_CH = _C // 2
_NCH = _EPW // _C

_MHI = np.int32(-65536)
_RND = np.int32(0x8000)


def _proj_body(src_ref, dst_ref, ws_ref, wd_ref, ps_ref, pd_ref):
    ps_ref[...] = jnp.dot(src_ref[...], ws_ref[...],
                          preferred_element_type=jnp.float32)
    pd_ref[...] = jnp.dot(dst_ref[...], wd_ref[...],
                          preferred_element_type=jnp.float32)


def _project(src, dst, ws, wd):
    bn = 2000
    return pl.pallas_call(
        _proj_body,
        grid=(N // bn,),
        in_specs=[
            pl.BlockSpec((bn, D), lambda i: (i, 0)),
            pl.BlockSpec((bn, D), lambda i: (i, 0)),
            pl.BlockSpec((D, H), lambda i: (0, 0)),
            pl.BlockSpec((D, H), lambda i: (0, 0)),
        ],
        out_specs=[
            pl.BlockSpec((bn, H), lambda i: (i, 0)),
            pl.BlockSpec((bn, H), lambda i: (i, 0)),
        ],
        out_shape=[
            jax.ShapeDtypeStruct((N, H), jnp.float32),
            jax.ShapeDtypeStruct((N, H), jnp.float32),
        ],
    )(src, dst, ws, wd)


@functools.partial(
    pl.kernel,
    out_type=jax.ShapeDtypeStruct((E // 2, H), jnp.float32),
    mesh=plsc.VectorSubcoreMesh(core_axis_name="c", subcore_axis_name="s"),
    scratch_types=[
        pltpu.VMEM((_C,), jnp.int32),
        pltpu.VMEM((_C,), jnp.int32),
        pltpu.VMEM((_C,), jnp.int32),
        pltpu.VMEM((_C,), jnp.int32),
        pltpu.VMEM((_C, H), jnp.float32),
        pltpu.VMEM((_C, H), jnp.float32),
        pltpu.VMEM((_C, H), jnp.float32),
        pltpu.VMEM((_C, H), jnp.float32),
        pltpu.SemaphoreType.DMA,
        pltpu.SemaphoreType.DMA,
        pltpu.SemaphoreType.DMA,
        pltpu.SemaphoreType.DMA,
        pltpu.SemaphoreType.DMA,
        pltpu.SemaphoreType.DMA,
    ],
)
def _sc_gather_sum(ps_hbm, pd_hbm, si_hbm, di_hbm, out_hbm,
                   idx_s0, idx_d0, idx_s1, idx_d1,
                   buf_a0, buf_b0, buf_a1, buf_b1,
                   sem_g0, sem_g1, sem_o0, sem_o1,
                   sem_i0, sem_i1):
    wid = lax.axis_index("s") * _NC + lax.axis_index("c")
    base = wid * _EPW
    base2 = wid * (_EPW // 2)
    idxs_s = (idx_s0, idx_s1)
    idxs_d = (idx_d0, idx_d1)
    bufs_a = (buf_a0, buf_a1)
    bufs_b = (buf_b0, buf_b1)
    sems_g = (sem_g0, sem_g1)
    sems_o = (sem_o0, sem_o1)
    sems_i = (sem_i0, sem_i1)

    def fire_idx(ci, b):
        off1 = base + ci * _CH
        off2 = base + _EPW // 2 + ci * _CH
        pltpu.async_copy(si_hbm.at[pl.ds(off1, _CH)],
                         idxs_s[b].at[pl.ds(0, _CH)], sems_i[b])
        pltpu.async_copy(si_hbm.at[pl.ds(off2, _CH)],
                         idxs_s[b].at[pl.ds(_CH, _CH)], sems_i[b])
        pltpu.async_copy(di_hbm.at[pl.ds(off1, _CH)],
                         idxs_d[b].at[pl.ds(0, _CH)], sems_i[b])
        pltpu.async_copy(di_hbm.at[pl.ds(off2, _CH)],
                         idxs_d[b].at[pl.ds(_CH, _CH)], sems_i[b])

    def drain_idx(b):
        for _ in range(2):
            pltpu.make_async_copy(si_hbm.at[pl.ds(0, _CH)],
                                  idxs_s[b].at[pl.ds(0, _CH)],
                                  sems_i[b]).wait()
            pltpu.make_async_copy(si_hbm.at[pl.ds(0, _CH)],
                                  idxs_d[b].at[pl.ds(0, _CH)],
                                  sems_i[b]).wait()

    def fire(ci, b):
        pltpu.async_copy(ps_hbm.at[idxs_s[b]], bufs_a[b], sems_g[b])
        pltpu.async_copy(pd_hbm.at[idxs_d[b]], bufs_b[b], sems_g[b])

    def drain_gather(b):
        pltpu.make_async_copy(ps_hbm.at[pl.ds(0, _C)], bufs_a[b],
                              sems_g[b]).wait()
        pltpu.make_async_copy(ps_hbm.at[pl.ds(0, _C)], bufs_b[b],
                              sems_g[b]).wait()

    def add_rows(b):
        ba, bb = bufs_a[b], bufs_b[b]

        def pack_row(src_row, dst_row, dst_col):
            for j in range(H // 32):
                slo = pl.ds(j * 32, 16)
                shi = pl.ds(j * 32 + 16, 16)
                se = ba[src_row, slo] + bb[src_row, slo]
                so = ba[src_row, shi] + bb[src_row, shi]
                ie = lax.bitcast_convert_type(se, jnp.int32)
                io = lax.bitcast_convert_type(so, jnp.int32)
                lo = lax.shift_right_logical(ie + _RND, 16)
                hi = (io + _RND) & _MHI
                ba[dst_row, pl.ds(dst_col + j * 16, 16)] = \
                    lax.bitcast_convert_type(hi | lo, jnp.float32)

        def row_body(r, c2):
            pack_row(r, r, 0)
            return c2

        def row_body2(r, c2):
            pack_row(r + _CH, r, H // 2)
            return c2

        lax.fori_loop(0, _CH, row_body, 0)
        lax.fori_loop(0, _CH, row_body2, 0)

    def start_out(ci, b):
        off2 = pl.multiple_of(base2 + ci * _CH, 8)
        pltpu.async_copy(bufs_a[b].at[pl.ds(0, _CH)],
                         out_hbm.at[pl.ds(off2, _CH)],
                         sems_o[b])

    def drain_out(b):
        pltpu.make_async_copy(ps_hbm.at[pl.ds(0, _CH)],
                              bufs_a[b].at[pl.ds(0, _CH)],
                              sems_o[b]).wait()

    fire_idx(0, 0)
    fire_idx(1, 1)
    drain_idx(0)
    fire(0, 0)
    drain_idx(1)
    fire(1, 1)

    def pair_body(g, carry):
        ci0 = 2 * g
        for b in range(2):
            ci = ci0 + b
            drain_gather(b)

            @pl.when(ci + 2 < _NCH)
            def _():
                fire_idx(ci + 2, b)

            add_rows(b)
            start_out(ci, b)

            @pl.when(ci + 2 < _NCH)
            def _():
                drain_idx(b)
                drain_out(b)
                fire(ci + 2, b)
        return carry

    lax.fori_loop(0, _NCH // 2, pair_body, 0)
    drain_gather(0)
    add_rows(0)
    start_out(_NCH - 1, 0)
    drain_out(0)
    drain_out(1)


_WPE = _EPW // 2


def _mlp_ln_t(x, ef, wet, b1, w2t, b2, g, bt):
    x = x + b1 + jnp.dot(wet, ef, preferred_element_type=jnp.float32)
    h = x * (1.0 / (1.0 + jnp.exp(-x)))
    y = jnp.dot(w2t, h, preferred_element_type=jnp.float32) + b2
    mu = jnp.mean(y, axis=0, keepdims=True)
    var = jnp.mean((y - mu) * (y - mu), axis=0, keepdims=True)
    return (y - mu) * lax.rsqrt(var + 1e-5) * g + bt + ef


def _tail_body(pk_ref, ef_ref, wet_ref, b1_ref, w2t_ref, b2_ref,
               g_ref, bt_ref, out_ref):
    wi = lax.bitcast_convert_type(jnp.transpose(pk_ref[...]), jnp.int32)
    lo = lax.bitcast_convert_type(wi << 16, jnp.float32)
    hi = lax.bitcast_convert_type(wi & _MHI, jnp.float32)
    xf = jnp.concatenate([lo[:H // 2], hi[:H // 2]], axis=0)
    xs = jnp.concatenate([lo[H // 2:], hi[H // 2:]], axis=0)
    ef_blk = ef_ref[...].reshape(DE, _EPW)
    args = (wet_ref[...], b1_ref[...], w2t_ref[...], b2_ref[...],
            g_ref[...], bt_ref[...])
    yf = _mlp_ln_t(xf, ef_blk[:, :_WPE], *args)
    ys = _mlp_ln_t(xs, ef_blk[:, _WPE:], *args)
    out_ref[...] = jnp.concatenate([yf, ys], axis=1).reshape(DE, 1, 1, _EPW)


def _tail(pk, ef_t, wet, b1, w2t, b2, gamma, beta):
    return pl.pallas_call(
        _tail_body,
        grid=(_NW,),
        in_specs=[
            pl.BlockSpec((_WPE, H), lambda i: (i, 0)),
            pl.BlockSpec((DE, 1, 1, _EPW), lambda i: (0, i, 0, 0)),
            pl.BlockSpec((H, DE), lambda i: (0, 0)),
            pl.BlockSpec((H, 1), lambda i: (0, 0)),
            pl.BlockSpec((DE, H), lambda i: (0, 0)),
            pl.BlockSpec((DE, 1), lambda i: (0, 0)),
            pl.BlockSpec((DE, 1), lambda i: (0, 0)),
            pl.BlockSpec((DE, 1), lambda i: (0, 0)),
        ],
        out_specs=pl.BlockSpec((DE, 1, 1, _EPW), lambda i: (0, i, 0, 0)),
        out_shape=jax.ShapeDtypeStruct((DE, _NW, 1, _EPW), jnp.float32),
    )(pk, ef_t, wet, b1, w2t, b2, gamma, beta)


_PERM = np.empty((H,), dtype=np.int32)
for _c in range(H // 2):
    _PERM[_c] = 32 * (_c // 16) + (_c % 16)
    _PERM[_c + H // 2] = _PERM[_c] + 16


def kernel(src_node_features, dst_node_features, edge_features,
           src_indices, dst_indices, W1, b1, W2, b2, gamma, beta):
    ws = W1[:D]
    wd = W1[D:2 * D]
    we = W1[2 * D:]
    si = src_indices.astype(jnp.int32)
    di = dst_indices.astype(jnp.int32)
    ps, pd = _project(src_node_features, dst_node_features, ws, wd)
    pk = _sc_gather_sum(ps, pd, si, di)
    ef_t = edge_features.T.reshape(DE, _NW, 1, _EPW)
    out_t = _tail(pk, ef_t, we[:, _PERM].T,
                  b1[_PERM].reshape(H, 1), W2[_PERM].T, b2.reshape(DE, 1),
                  gamma.reshape(DE, 1), beta.reshape(DE, 1))
    return out_t.reshape(DE, E).T


# speedup vs baseline: 59.5983x; 1.6073x over previous
import functools

import numpy as np

import jax
import jax.numpy as jnp
from jax import lax
from jax.experimental import pallas as pl
from jax.experimental.pallas import tpu as pltpu
from jax.experimental.pallas import tpu_sc as plsc

N = 10000
E = 320000
D = 128
DE = 16
H = 128

_info = plsc.get_sparse_core_info()
_NC = _info.num_cores
_NS = _info.num_subcores
_NW = _NC * _NS
_EPW = E // _NW
_C = 80
_CH = _C // 2
_NCH = _EPW // _C

_MHI = np.int32(-65536)
_RND = np.int32(0x8000)


def _proj_body(src_ref, dst_ref, ws_ref, wd_ref, ps_ref, pd_ref):
    ps_ref[...] = jnp.dot(src_ref[...], ws_ref[...],
                          preferred_element_type=jnp.float32)
    pd_ref[...] = jnp.dot(dst_ref[...], wd_ref[...],
                          preferred_element_type=jnp.float32)


def _project(src, dst, ws, wd):
    bn = 2000
    return pl.pallas_call(
        _proj_body,
        grid=(N // bn,),
        in_specs=[
            pl.BlockSpec((bn, D), lambda i: (i, 0)),
            pl.BlockSpec((bn, D), lambda i: (i, 0)),
            pl.BlockSpec((D, H), lambda i: (0, 0)),
            pl.BlockSpec((D, H), lambda i: (0, 0)),
        ],
        out_specs=[
            pl.BlockSpec((bn, H), lambda i: (i, 0)),
            pl.BlockSpec((bn, H), lambda i: (i, 0)),
        ],
        out_shape=[
            jax.ShapeDtypeStruct((N, H), jnp.float32),
            jax.ShapeDtypeStruct((N, H), jnp.float32),
        ],
    )(src, dst, ws, wd)


@functools.partial(
    pl.kernel,
    out_type=jax.ShapeDtypeStruct((E // 2, H), jnp.float32),
    mesh=plsc.VectorSubcoreMesh(core_axis_name="c", subcore_axis_name="s"),
    scratch_types=[
        pltpu.VMEM((_C,), jnp.int32),
        pltpu.VMEM((_C,), jnp.int32),
        pltpu.VMEM((_C,), jnp.int32),
        pltpu.VMEM((_C,), jnp.int32),
        pltpu.VMEM((_C, H), jnp.float32),
        pltpu.VMEM((_C, H), jnp.float32),
        pltpu.VMEM((_C, H), jnp.float32),
        pltpu.VMEM((_C, H), jnp.float32),
        pltpu.SemaphoreType.DMA,
        pltpu.SemaphoreType.DMA,
        pltpu.SemaphoreType.DMA,
        pltpu.SemaphoreType.DMA,
        pltpu.SemaphoreType.DMA,
        pltpu.SemaphoreType.DMA,
    ],
)
def _sc_gather_sum(ps_hbm, pd_hbm, si_hbm, di_hbm, out_hbm,
                   idx_s0, idx_d0, idx_s1, idx_d1,
                   buf_a0, buf_b0, buf_a1, buf_b1,
                   sem_g0, sem_g1, sem_o0, sem_o1,
                   sem_i0, sem_i1):
    wid = lax.axis_index("s") * _NC + lax.axis_index("c")
    base = wid * _EPW
    base2 = wid * (_EPW // 2)
    idxs_s = (idx_s0, idx_s1)
    idxs_d = (idx_d0, idx_d1)
    bufs_a = (buf_a0, buf_a1)
    bufs_b = (buf_b0, buf_b1)
    sems_g = (sem_g0, sem_g1)
    sems_o = (sem_o0, sem_o1)
    sems_i = (sem_i0, sem_i1)

    def fire_idx(ci, b):
        off1 = base2 + ci * _CH
        off2 = E // 2 + base2 + ci * _CH
        pltpu.async_copy(si_hbm.at[pl.ds(off1, _CH)],
                         idxs_s[b].at[pl.ds(0, _CH)], sems_i[b])
        pltpu.async_copy(si_hbm.at[pl.ds(off2, _CH)],
                         idxs_s[b].at[pl.ds(_CH, _CH)], sems_i[b])
        pltpu.async_copy(di_hbm.at[pl.ds(off1, _CH)],
                         idxs_d[b].at[pl.ds(0, _CH)], sems_i[b])
        pltpu.async_copy(di_hbm.at[pl.ds(off2, _CH)],
                         idxs_d[b].at[pl.ds(_CH, _CH)], sems_i[b])

    def drain_idx(b):
        for _ in range(2):
            pltpu.make_async_copy(si_hbm.at[pl.ds(0, _CH)],
                                  idxs_s[b].at[pl.ds(0, _CH)],
                                  sems_i[b]).wait()
            pltpu.make_async_copy(si_hbm.at[pl.ds(0, _CH)],
                                  idxs_d[b].at[pl.ds(0, _CH)],
                                  sems_i[b]).wait()

    def fire(ci, b):
        pltpu.async_copy(ps_hbm.at[idxs_s[b]], bufs_a[b], sems_g[b])
        pltpu.async_copy(pd_hbm.at[idxs_d[b]], bufs_b[b], sems_g[b])

    def drain_gather(b):
        pltpu.make_async_copy(ps_hbm.at[pl.ds(0, _C)], bufs_a[b],
                              sems_g[b]).wait()
        pltpu.make_async_copy(ps_hbm.at[pl.ds(0, _C)], bufs_b[b],
                              sems_g[b]).wait()

    def add_rows(b):
        ba, bb = bufs_a[b], bufs_b[b]

        def pack_row(src_row, dst_row, dst_col):
            for j in range(H // 32):
                slo = pl.ds(j * 32, 16)
                shi = pl.ds(j * 32 + 16, 16)
                se = ba[src_row, slo] + bb[src_row, slo]
                so = ba[src_row, shi] + bb[src_row, shi]
                ie = lax.bitcast_convert_type(se, jnp.int32)
                io = lax.bitcast_convert_type(so, jnp.int32)
                lo = lax.shift_right_logical(ie + _RND, 16)
                hi = (io + _RND) & _MHI
                ba[dst_row, pl.ds(dst_col + j * 16, 16)] = \
                    lax.bitcast_convert_type(hi | lo, jnp.float32)

        def row_body(r, c2):
            pack_row(r, r, 0)
            return c2

        def row_body2(r, c2):
            pack_row(r + _CH, r, H // 2)
            return c2

        lax.fori_loop(0, _CH, row_body, 0)
        lax.fori_loop(0, _CH, row_body2, 0)

    def start_out(ci, b):
        off2 = pl.multiple_of(base2 + ci * _CH, 8)
        pltpu.async_copy(bufs_a[b].at[pl.ds(0, _CH)],
                         out_hbm.at[pl.ds(off2, _CH)],
                         sems_o[b])

    def drain_out(b):
        pltpu.make_async_copy(ps_hbm.at[pl.ds(0, _CH)],
                              bufs_a[b].at[pl.ds(0, _CH)],
                              sems_o[b]).wait()

    fire_idx(0, 0)
    fire_idx(1, 1)
    drain_idx(0)
    fire(0, 0)
    drain_idx(1)
    fire(1, 1)

    def pair_body(g, carry):
        ci0 = 2 * g
        for b in range(2):
            ci = ci0 + b
            drain_gather(b)

            @pl.when(ci + 2 < _NCH)
            def _():
                fire_idx(ci + 2, b)

            add_rows(b)
            start_out(ci, b)

            @pl.when(ci + 2 < _NCH)
            def _():
                drain_idx(b)
                drain_out(b)
                fire(ci + 2, b)
        return carry

    lax.fori_loop(0, _NCH // 2, pair_body, 0)
    drain_gather(0)
    add_rows(0)
    start_out(_NCH - 1, 0)
    drain_out(0)
    drain_out(1)


_BP = 6400


def _mlp_ln_t(x, ef, wet, b1, w2t, b2, g, bt):
    x = x + b1 + jnp.dot(wet, ef, preferred_element_type=jnp.float32)
    h = x * (1.0 / (1.0 + jnp.exp(-x)))
    y = jnp.dot(w2t, h, preferred_element_type=jnp.float32) + b2
    mu = jnp.mean(y, axis=0, keepdims=True)
    var = jnp.mean((y - mu) * (y - mu), axis=0, keepdims=True)
    return (y - mu) * lax.rsqrt(var + 1e-5) * g + bt + ef


def _tail_body(pk_ref, ef_ref, wet_ref, b1_ref, w2t_ref, b2_ref,
               g_ref, bt_ref, out_ref):
    hh = pl.program_id(1)
    args = (wet_ref[...], b1_ref[...], w2t_ref[...], b2_ref[...],
            g_ref[...], bt_ref[...])

    def half(sel):
        cols = pk_ref[...][:, sel * (H // 2):(sel + 1) * (H // 2)]
        wi = lax.bitcast_convert_type(jnp.transpose(cols), jnp.int32)
        lo = lax.bitcast_convert_type(wi << 16, jnp.float32)
        hi = lax.bitcast_convert_type(wi & _MHI, jnp.float32)
        x = jnp.concatenate([lo, hi], axis=0)
        out_ref[...] = _mlp_ln_t(x, ef_ref[...], *args)

    @pl.when(hh == 0)
    def _():
        half(0)

    @pl.when(hh == 1)
    def _():
        half(1)


def _tail(pk, ef_t, wet, b1, w2t, b2, gamma, beta):
    nb = (E // 2) // _BP
    return pl.pallas_call(
        _tail_body,
        grid=(nb, 2),
        in_specs=[
            pl.BlockSpec((_BP, H), lambda i, h: (i, 0)),
            pl.BlockSpec((DE, _BP), lambda i, h: (0, i + nb * h)),
            pl.BlockSpec((H, DE), lambda i, h: (0, 0)),
            pl.BlockSpec((H, 1), lambda i, h: (0, 0)),
            pl.BlockSpec((DE, H), lambda i, h: (0, 0)),
            pl.BlockSpec((DE, 1), lambda i, h: (0, 0)),
            pl.BlockSpec((DE, 1), lambda i, h: (0, 0)),
            pl.BlockSpec((DE, 1), lambda i, h: (0, 0)),
        ],
        out_specs=pl.BlockSpec((DE, _BP), lambda i, h: (0, i + nb * h)),
        out_shape=jax.ShapeDtypeStruct((DE, E), jnp.float32),
    )(pk, ef_t, wet, b1, w2t, b2, gamma, beta)


_PERM = np.empty((H,), dtype=np.int32)
for _c in range(H // 2):
    _PERM[_c] = 32 * (_c // 16) + (_c % 16)
    _PERM[_c + H // 2] = _PERM[_c] + 16


def kernel(src_node_features, dst_node_features, edge_features,
           src_indices, dst_indices, W1, b1, W2, b2, gamma, beta):
    ws = W1[:D]
    wd = W1[D:2 * D]
    we = W1[2 * D:]
    si = src_indices.astype(jnp.int32)
    di = dst_indices.astype(jnp.int32)
    ps, pd = _project(src_node_features, dst_node_features, ws, wd)
    pk = _sc_gather_sum(ps, pd, si, di)
    out_t = _tail(pk, edge_features.T, we[:, _PERM].T,
                  b1[_PERM].reshape(H, 1), W2[_PERM].T, b2.reshape(DE, 1),
                  gamma.reshape(DE, 1), beta.reshape(DE, 1))
    return out_t.T


# speedup vs baseline: 60.5747x; 1.0164x over previous
import functools

import numpy as np

import jax
import jax.numpy as jnp
from jax import lax
from jax.experimental import pallas as pl
from jax.experimental.pallas import tpu as pltpu
from jax.experimental.pallas import tpu_sc as plsc

N = 10000
E = 320000
D = 128
DE = 16
H = 128

_info = plsc.get_sparse_core_info()
_NC = _info.num_cores
_NS = _info.num_subcores
_NW = _NC * _NS
_EPW = E // _NW
_C = 80
_CH = _C // 2
_NCH = _EPW // _C

_MHI = np.int32(-65536)
_RND = np.int32(0x8000)


def _proj_body(src_ref, dst_ref, ws_ref, wd_ref, ps_ref, pd_ref):
    ps_ref[...] = jnp.dot(src_ref[...], ws_ref[...],
                          preferred_element_type=jnp.float32)
    pd_ref[...] = jnp.dot(dst_ref[...], wd_ref[...],
                          preferred_element_type=jnp.float32)


def _project(src, dst, ws, wd):
    bn = 2000
    return pl.pallas_call(
        _proj_body,
        grid=(N // bn,),
        in_specs=[
            pl.BlockSpec((bn, D), lambda i: (i, 0)),
            pl.BlockSpec((bn, D), lambda i: (i, 0)),
            pl.BlockSpec((D, H), lambda i: (0, 0)),
            pl.BlockSpec((D, H), lambda i: (0, 0)),
        ],
        out_specs=[
            pl.BlockSpec((bn, H), lambda i: (i, 0)),
            pl.BlockSpec((bn, H), lambda i: (i, 0)),
        ],
        out_shape=[
            jax.ShapeDtypeStruct((N, H), jnp.float32),
            jax.ShapeDtypeStruct((N, H), jnp.float32),
        ],
    )(src, dst, ws, wd)


@functools.partial(
    pl.kernel,
    out_type=jax.ShapeDtypeStruct((E // 2, H), jnp.float32),
    mesh=plsc.VectorSubcoreMesh(core_axis_name="c", subcore_axis_name="s"),
    scratch_types=[
        pltpu.VMEM((_C,), jnp.int32),
        pltpu.VMEM((_C,), jnp.int32),
        pltpu.VMEM((_C,), jnp.int32),
        pltpu.VMEM((_C,), jnp.int32),
        pltpu.VMEM((_C, H), jnp.float32),
        pltpu.VMEM((_C, H), jnp.float32),
        pltpu.VMEM((_C, H), jnp.float32),
        pltpu.VMEM((_C, H), jnp.float32),
        pltpu.SemaphoreType.DMA,
        pltpu.SemaphoreType.DMA,
        pltpu.SemaphoreType.DMA,
        pltpu.SemaphoreType.DMA,
        pltpu.SemaphoreType.DMA,
        pltpu.SemaphoreType.DMA,
    ],
)
def _sc_gather_sum(ps_hbm, pd_hbm, si_hbm, di_hbm, out_hbm,
                   idx_s0, idx_d0, idx_s1, idx_d1,
                   buf_a0, buf_b0, buf_a1, buf_b1,
                   sem_g0, sem_g1, sem_o0, sem_o1,
                   sem_i0, sem_i1):
    wid = lax.axis_index("s") * _NC + lax.axis_index("c")
    base = wid * _EPW
    base2 = wid * (_EPW // 2)
    idxs_s = (idx_s0, idx_s1)
    idxs_d = (idx_d0, idx_d1)
    bufs_a = (buf_a0, buf_a1)
    bufs_b = (buf_b0, buf_b1)
    sems_g = (sem_g0, sem_g1)
    sems_o = (sem_o0, sem_o1)
    sems_i = (sem_i0, sem_i1)

    def fire_idx(ci, b):
        off1 = base2 + ci * _CH
        off2 = E // 2 + base2 + ci * _CH
        pltpu.async_copy(si_hbm.at[pl.ds(off1, _CH)],
                         idxs_s[b].at[pl.ds(0, _CH)], sems_i[b])
        pltpu.async_copy(si_hbm.at[pl.ds(off2, _CH)],
                         idxs_s[b].at[pl.ds(_CH, _CH)], sems_i[b])
        pltpu.async_copy(di_hbm.at[pl.ds(off1, _CH)],
                         idxs_d[b].at[pl.ds(0, _CH)], sems_i[b])
        pltpu.async_copy(di_hbm.at[pl.ds(off2, _CH)],
                         idxs_d[b].at[pl.ds(_CH, _CH)], sems_i[b])

    def drain_idx(b):
        for _ in range(2):
            pltpu.make_async_copy(si_hbm.at[pl.ds(0, _CH)],
                                  idxs_s[b].at[pl.ds(0, _CH)],
                                  sems_i[b]).wait()
            pltpu.make_async_copy(si_hbm.at[pl.ds(0, _CH)],
                                  idxs_d[b].at[pl.ds(0, _CH)],
                                  sems_i[b]).wait()

    def fire(ci, b):
        pltpu.async_copy(ps_hbm.at[idxs_s[b]], bufs_a[b], sems_g[b])
        pltpu.async_copy(pd_hbm.at[idxs_d[b]], bufs_b[b], sems_g[b])

    def drain_gather(b):
        pltpu.make_async_copy(ps_hbm.at[pl.ds(0, _C)], bufs_a[b],
                              sems_g[b]).wait()
        pltpu.make_async_copy(ps_hbm.at[pl.ds(0, _C)], bufs_b[b],
                              sems_g[b]).wait()

    def add_rows(b):
        ba, bb = bufs_a[b], bufs_b[b]

        def pack_row(src_row, dst_row, dst_col):
            for j in range(H // 32):
                slo = pl.ds(j * 32, 16)
                shi = pl.ds(j * 32 + 16, 16)
                se = ba[src_row, slo] + bb[src_row, slo]
                so = ba[src_row, shi] + bb[src_row, shi]
                ie = lax.bitcast_convert_type(se, jnp.int32)
                io = lax.bitcast_convert_type(so, jnp.int32)
                lo = lax.shift_right_logical(ie + _RND, 16)
                hi = (io + _RND) & _MHI
                ba[dst_row, pl.ds(dst_col + j * 16, 16)] = \
                    lax.bitcast_convert_type(hi | lo, jnp.float32)

        def row_body(r, c2):
            pack_row(r, r, 0)
            return c2

        def row_body2(r, c2):
            pack_row(r + _CH, r, H // 2)
            return c2

        lax.fori_loop(0, _CH, row_body, 0)
        lax.fori_loop(0, _CH, row_body2, 0)

    def start_out(ci, b):
        off2 = pl.multiple_of(base2 + ci * _CH, 8)
        pltpu.async_copy(bufs_a[b].at[pl.ds(0, _CH)],
                         out_hbm.at[pl.ds(off2, _CH)],
                         sems_o[b])

    def drain_out(b):
        pltpu.make_async_copy(ps_hbm.at[pl.ds(0, _CH)],
                              bufs_a[b].at[pl.ds(0, _CH)],
                              sems_o[b]).wait()

    fire_idx(0, 0)
    fire_idx(1, 1)
    drain_idx(0)
    fire(0, 0)
    drain_idx(1)
    fire(1, 1)

    def pair_body(g, carry):
        ci0 = 2 * g
        for b in range(2):
            ci = ci0 + b
            drain_gather(b)

            @pl.when(ci + 2 < _NCH)
            def _():
                fire_idx(ci + 2, b)

            add_rows(b)
            start_out(ci, b)

            @pl.when(ci + 2 < _NCH)
            def _():
                drain_idx(b)
                drain_out(b)
                fire(ci + 2, b)
        return carry

    lax.fori_loop(0, _NCH // 2, pair_body, 0)
    drain_gather(0)
    add_rows(0)
    start_out(_NCH - 1, 0)
    drain_out(0)
    drain_out(1)


_BP = 6400


def _mlp_ln_t(x, ef, wet, b1, w2t, b2, g, bt):
    x = x + b1 + jnp.dot(wet, ef, preferred_element_type=jnp.float32)
    h = x * (0.5 + 0.5 * jnp.tanh(0.5 * x))
    y = jnp.dot(w2t, h, preferred_element_type=jnp.float32) + b2
    mu = jnp.mean(y, axis=0, keepdims=True)
    var = jnp.mean((y - mu) * (y - mu), axis=0, keepdims=True)
    return (y - mu) * lax.rsqrt(var + 1e-5) * g + bt + ef


def _tail_body(pk_ref, ef_ref, wet_ref, b1_ref, w2t_ref, b2_ref,
               g_ref, bt_ref, out_ref):
    hh = pl.program_id(1)
    args = (wet_ref[...], b1_ref[...], w2t_ref[...], b2_ref[...],
            g_ref[...], bt_ref[...])

    def half(sel):
        cols = pk_ref[...][:, sel * (H // 2):(sel + 1) * (H // 2)]
        wi = lax.bitcast_convert_type(jnp.transpose(cols), jnp.int32)
        lo = lax.bitcast_convert_type(wi << 16, jnp.float32)
        hi = lax.bitcast_convert_type(wi & _MHI, jnp.float32)
        x = jnp.concatenate([lo, hi], axis=0)
        out_ref[...] = _mlp_ln_t(x, ef_ref[...], *args)

    @pl.when(hh == 0)
    def _():
        half(0)

    @pl.when(hh == 1)
    def _():
        half(1)


def _tail(pk, ef_t, wet, b1, w2t, b2, gamma, beta):
    nb = (E // 2) // _BP
    return pl.pallas_call(
        _tail_body,
        grid=(nb, 2),
        in_specs=[
            pl.BlockSpec((_BP, H), lambda i, h: (i, 0)),
            pl.BlockSpec((DE, _BP), lambda i, h: (0, i + nb * h)),
            pl.BlockSpec((H, DE), lambda i, h: (0, 0)),
            pl.BlockSpec((H, 1), lambda i, h: (0, 0)),
            pl.BlockSpec((DE, H), lambda i, h: (0, 0)),
            pl.BlockSpec((DE, 1), lambda i, h: (0, 0)),
            pl.BlockSpec((DE, 1), lambda i, h: (0, 0)),
            pl.BlockSpec((DE, 1), lambda i, h: (0, 0)),
        ],
        out_specs=pl.BlockSpec((DE, _BP), lambda i, h: (0, i + nb * h)),
        out_shape=jax.ShapeDtypeStruct((DE, E), jnp.float32),
    )(pk, ef_t, wet, b1, w2t, b2, gamma, beta)


_PERM = np.empty((H,), dtype=np.int32)
for _c in range(H // 2):
    _PERM[_c] = 32 * (_c // 16) + (_c % 16)
    _PERM[_c + H // 2] = _PERM[_c] + 16


def kernel(src_node_features, dst_node_features, edge_features,
           src_indices, dst_indices, W1, b1, W2, b2, gamma, beta):
    ws = W1[:D]
    wd = W1[D:2 * D]
    we = W1[2 * D:]
    si = src_indices.astype(jnp.int32)
    di = dst_indices.astype(jnp.int32)
    ps, pd = _project(src_node_features, dst_node_features, ws, wd)
    pk = _sc_gather_sum(ps, pd, si, di)
    out_t = _tail(pk, edge_features.T, we[:, _PERM].T,
                  b1[_PERM].reshape(H, 1), W2[_PERM].T, b2.reshape(DE, 1),
                  gamma.reshape(DE, 1), beta.reshape(DE, 1))
    return out_t.T


# speedup vs baseline: 64.6441x; 1.0672x over previous
import functools

import numpy as np

import jax
import jax.numpy as jnp
from jax import lax
from jax.experimental import pallas as pl
from jax.experimental.pallas import tpu as pltpu
from jax.experimental.pallas import tpu_sc as plsc

N = 10000
E = 320000
D = 128
DE = 16
H = 128

_info = plsc.get_sparse_core_info()
_NC = _info.num_cores
_NS = _info.num_subcores
_NW = _NC * _NS
_EPW = E // _NW
_C = 80
_CH = _C // 2
_NSEG = 5
_ESEG = E // _NSEG
_PSEG = _ESEG // 2
_RPW = _PSEG // _NW
_NCH = _RPW // _CH

_MHI = np.int32(-65536)
_RND = np.int32(0x8000)


def _proj_body(src_ref, dst_ref, ws_ref, wd_ref, ps_ref, pd_ref):
    ps_ref[...] = jnp.dot(src_ref[...], ws_ref[...],
                          preferred_element_type=jnp.float32)
    pd_ref[...] = jnp.dot(dst_ref[...], wd_ref[...],
                          preferred_element_type=jnp.float32)


def _project(src, dst, ws, wd):
    bn = 2000
    return pl.pallas_call(
        _proj_body,
        grid=(N // bn,),
        in_specs=[
            pl.BlockSpec((bn, D), lambda i: (i, 0)),
            pl.BlockSpec((bn, D), lambda i: (i, 0)),
            pl.BlockSpec((D, H), lambda i: (0, 0)),
            pl.BlockSpec((D, H), lambda i: (0, 0)),
        ],
        out_specs=[
            pl.BlockSpec((bn, H), lambda i: (i, 0)),
            pl.BlockSpec((bn, H), lambda i: (i, 0)),
        ],
        out_shape=[
            jax.ShapeDtypeStruct((N, H), jnp.float32),
            jax.ShapeDtypeStruct((N, H), jnp.float32),
        ],
    )(src, dst, ws, wd)


def _make_sc(seg):
    ebase = seg * _ESEG

    @functools.partial(
        pl.kernel,
        out_type=jax.ShapeDtypeStruct((_PSEG, H), jnp.float32),
        mesh=plsc.VectorSubcoreMesh(core_axis_name="c", subcore_axis_name="s"),
        scratch_types=[
            pltpu.VMEM((_C,), jnp.int32),
            pltpu.VMEM((_C,), jnp.int32),
            pltpu.VMEM((_C,), jnp.int32),
            pltpu.VMEM((_C,), jnp.int32),
            pltpu.VMEM((_C, H), jnp.float32),
            pltpu.VMEM((_C, H), jnp.float32),
            pltpu.VMEM((_C, H), jnp.float32),
            pltpu.VMEM((_C, H), jnp.float32),
            pltpu.SemaphoreType.DMA,
            pltpu.SemaphoreType.DMA,
            pltpu.SemaphoreType.DMA,
            pltpu.SemaphoreType.DMA,
            pltpu.SemaphoreType.DMA,
            pltpu.SemaphoreType.DMA,
        ],
    )
    def _sc_gather_sum(ps_hbm, pd_hbm, si_hbm, di_hbm, out_hbm,
                       idx_s0, idx_d0, idx_s1, idx_d1,
                       buf_a0, buf_b0, buf_a1, buf_b1,
                       sem_g0, sem_g1, sem_o0, sem_o1,
                       sem_i0, sem_i1):
        wid = lax.axis_index("s") * _NC + lax.axis_index("c")
        base2 = wid * _RPW
        idxs_s = (idx_s0, idx_s1)
        idxs_d = (idx_d0, idx_d1)
        bufs_a = (buf_a0, buf_a1)
        bufs_b = (buf_b0, buf_b1)
        sems_g = (sem_g0, sem_g1)
        sems_o = (sem_o0, sem_o1)
        sems_i = (sem_i0, sem_i1)

        def fire_idx(ci, b):
            off1 = ebase + base2 + ci * _CH
            off2 = off1 + _ESEG // 2
            pltpu.async_copy(si_hbm.at[pl.ds(off1, _CH)],
                             idxs_s[b].at[pl.ds(0, _CH)], sems_i[b])
            pltpu.async_copy(si_hbm.at[pl.ds(off2, _CH)],
                             idxs_s[b].at[pl.ds(_CH, _CH)], sems_i[b])
            pltpu.async_copy(di_hbm.at[pl.ds(off1, _CH)],
                             idxs_d[b].at[pl.ds(0, _CH)], sems_i[b])
            pltpu.async_copy(di_hbm.at[pl.ds(off2, _CH)],
                             idxs_d[b].at[pl.ds(_CH, _CH)], sems_i[b])

        def drain_idx(b):
            for _ in range(2):
                pltpu.make_async_copy(si_hbm.at[pl.ds(0, _CH)],
                                      idxs_s[b].at[pl.ds(0, _CH)],
                                      sems_i[b]).wait()
                pltpu.make_async_copy(si_hbm.at[pl.ds(0, _CH)],
                                      idxs_d[b].at[pl.ds(0, _CH)],
                                      sems_i[b]).wait()

        def fire(ci, b):
            pltpu.async_copy(ps_hbm.at[idxs_s[b]], bufs_a[b], sems_g[b])
            pltpu.async_copy(pd_hbm.at[idxs_d[b]], bufs_b[b], sems_g[b])

        def drain_gather(b):
            pltpu.make_async_copy(ps_hbm.at[pl.ds(0, _C)], bufs_a[b],
                                  sems_g[b]).wait()
            pltpu.make_async_copy(ps_hbm.at[pl.ds(0, _C)], bufs_b[b],
                                  sems_g[b]).wait()

        def add_rows(b):
            ba, bb = bufs_a[b], bufs_b[b]

            def pack_row(src_row, dst_row, dst_col):
                for j in range(H // 32):
                    slo = pl.ds(j * 32, 16)
                    shi = pl.ds(j * 32 + 16, 16)
                    se = ba[src_row, slo] + bb[src_row, slo]
                    so = ba[src_row, shi] + bb[src_row, shi]
                    ie = lax.bitcast_convert_type(se, jnp.int32)
                    io = lax.bitcast_convert_type(so, jnp.int32)
                    lo = lax.shift_right_logical(ie + _RND, 16)
                    hi = (io + _RND) & _MHI
                    ba[dst_row, pl.ds(dst_col + j * 16, 16)] = \
                        lax.bitcast_convert_type(hi | lo, jnp.float32)

            def row_body(r, c2):
                pack_row(r, r, 0)
                return c2

            def row_body2(r, c2):
                pack_row(r + _CH, r, H // 2)
                return c2

            lax.fori_loop(0, _CH, row_body, 0)
            lax.fori_loop(0, _CH, row_body2, 0)

        def start_out(ci, b):
            off2 = pl.multiple_of(base2 + ci * _CH, 8)
            pltpu.async_copy(bufs_a[b].at[pl.ds(0, _CH)],
                             out_hbm.at[pl.ds(off2, _CH)],
                             sems_o[b])

        def drain_out(b):
            pltpu.make_async_copy(ps_hbm.at[pl.ds(0, _CH)],
                                  bufs_a[b].at[pl.ds(0, _CH)],
                                  sems_o[b]).wait()

        fire_idx(0, 0)
        fire_idx(1, 1)
        drain_idx(0)
        fire(0, 0)
        drain_idx(1)
        fire(1, 1)

        def pair_body(g, carry):
            ci0 = 2 * g
            for b in range(2):
                ci = ci0 + b
                drain_gather(b)

                @pl.when(ci + 2 < _NCH)
                def _():
                    fire_idx(ci + 2, b)

                add_rows(b)
                start_out(ci, b)

                @pl.when(ci + 2 < _NCH)
                def _():
                    drain_idx(b)
                    drain_out(b)
                    fire(ci + 2, b)
            return carry

        lax.fori_loop(0, _NCH // 2, pair_body, 0)
        drain_gather(0)
        add_rows(0)
        start_out(_NCH - 1, 0)
        drain_out(0)
        drain_out(1)

    return _sc_gather_sum


_SC_SEGS = tuple(_make_sc(k) for k in range(_NSEG))


_BP = 6400


def _mlp_ln_t(x, ef, wet, b1, w2t, b2, g, bt):
    x = x + b1 + jnp.dot(wet, ef, preferred_element_type=jnp.float32)
    h = x * (0.5 + 0.5 * jnp.tanh(0.5 * x))
    y = jnp.dot(w2t, h, preferred_element_type=jnp.float32) + b2
    mu = jnp.mean(y, axis=0, keepdims=True)
    var = jnp.mean((y - mu) * (y - mu), axis=0, keepdims=True)
    return (y - mu) * lax.rsqrt(var + 1e-5) * g + bt + ef


def _tail_body(pk_ref, ef_ref, wet_ref, b1_ref, w2t_ref, b2_ref,
               g_ref, bt_ref, out_ref):
    hh = pl.program_id(1)
    args = (wet_ref[...], b1_ref[...], w2t_ref[...], b2_ref[...],
            g_ref[...], bt_ref[...])

    def half(sel):
        cols = pk_ref[...][:, sel * (H // 2):(sel + 1) * (H // 2)]
        wi = lax.bitcast_convert_type(jnp.transpose(cols), jnp.int32)
        lo = lax.bitcast_convert_type(wi << 16, jnp.float32)
        hi = lax.bitcast_convert_type(wi & _MHI, jnp.float32)
        x = jnp.concatenate([lo, hi], axis=0)
        out_ref[...] = _mlp_ln_t(x, ef_ref[...], *args)

    @pl.when(hh == 0)
    def _():
        half(0)

    @pl.when(hh == 1)
    def _():
        half(1)


def _tail_seg(seg, pk, ef_t, wet, b1, w2t, b2, gamma, beta):
    nb = _PSEG // _BP
    ebase = seg * _ESEG // _BP
    return pl.pallas_call(
        _tail_body,
        grid=(nb, 2),
        in_specs=[
            pl.BlockSpec((_BP, H), lambda i, h: (i, 0)),
            pl.BlockSpec((DE, _BP), lambda i, h: (0, ebase + i + nb * h)),
            pl.BlockSpec((H, DE), lambda i, h: (0, 0)),
            pl.BlockSpec((H, 1), lambda i, h: (0, 0)),
            pl.BlockSpec((DE, H), lambda i, h: (0, 0)),
            pl.BlockSpec((DE, 1), lambda i, h: (0, 0)),
            pl.BlockSpec((DE, 1), lambda i, h: (0, 0)),
            pl.BlockSpec((DE, 1), lambda i, h: (0, 0)),
        ],
        out_specs=pl.BlockSpec((DE, _BP), lambda i, h: (0, i + nb * h)),
        out_shape=jax.ShapeDtypeStruct((DE, _ESEG), jnp.float32),
    )(pk, ef_t, wet, b1, w2t, b2, gamma, beta)


_PERM = np.empty((H,), dtype=np.int32)
for _c in range(H // 2):
    _PERM[_c] = 32 * (_c // 16) + (_c % 16)
    _PERM[_c + H // 2] = _PERM[_c] + 16


def kernel(src_node_features, dst_node_features, edge_features,
           src_indices, dst_indices, W1, b1, W2, b2, gamma, beta):
    ws = W1[:D]
    wd = W1[D:2 * D]
    we = W1[2 * D:]
    si = src_indices.astype(jnp.int32)
    di = dst_indices.astype(jnp.int32)
    ps, pd = _project(src_node_features, dst_node_features, ws, wd)
    ef_t = edge_features.T
    wargs = (we[:, _PERM].T, b1[_PERM].reshape(H, 1), W2[_PERM].T,
             b2.reshape(DE, 1), gamma.reshape(DE, 1), beta.reshape(DE, 1))
    outs = []
    for k in range(_NSEG):
        pk = _SC_SEGS[k](ps, pd, si, di)
        outs.append(_tail_seg(k, pk, ef_t, *wargs))
    return jnp.concatenate(outs, axis=1).T
